# baseline (speedup 1.0000x reference)
import functools

import jax
import jax.numpy as jnp
from jax.experimental import pallas as pl
from jax.experimental.pallas import tpu as pltpu

TM = 1024
NBUF = 3
DEPTH = 2


def _body(a_hbm, h_hbm, w_ref, b_ref, o_hbm,
          a_buf, h_vmem, hw_ref, b2_ref, o_buf, a_sem, h_sem, o_sem,
          *, B, N, L, D):
    nt = N // TM
    T = B * nt
    LD = L * D

    def a_copy(t):
        bt, it = t // nt, t % nt
        return pltpu.make_async_copy(
            a_hbm.at[bt, pl.ds(it * TM, TM), :], a_buf.at[t % NBUF],
            a_sem.at[t % NBUF])

    def h_copy(bt):
        return pltpu.make_async_copy(
            h_hbm.at[bt], h_vmem.at[bt % 2], h_sem.at[bt % 2])

    def o_copy(t):
        bt, it = t // nt, t % nt
        return pltpu.make_async_copy(
            o_buf.at[t % 2], o_hbm.at[bt, pl.ds(it * TM, TM), :],
            o_sem.at[t % 2])

    for ll in range(L):
        b2_ref[0, ll * D:(ll + 1) * D] = b_ref[0]

    h_copy(0).start()
    for t in range(min(DEPTH, T)):
        a_copy(t).start()

    for t in range(T):
        bt, it = t // nt, t % nt
        if it == 0:
            h_copy(bt).wait()
            if bt + 1 < B:
                h_copy(bt + 1).start()
            h = h_vmem[bt % 2]
            for ll in range(L):
                hw_ref[bt % 2, :, ll * D:(ll + 1) * D] = jax.lax.dot_general(
                    h[:, ll * D:(ll + 1) * D], w_ref[...],
                    (((1,), (1,)), ((), ())),
                    preferred_element_type=jnp.float32)
        a_copy(t).wait()
        if t >= 2:
            o_copy(t - 2).wait()
        out = jnp.dot(a_buf[t % NBUF], hw_ref[bt % 2],
                      preferred_element_type=jnp.float32)
        o_buf[t % 2] = jnp.maximum(out + b2_ref[...], 0.0)
        o_copy(t).start()
        if t + DEPTH < T:
            a_copy(t + DEPTH).start()

    for t in range(max(T - 2, 0), T):
        o_copy(t).wait()


def kernel(prop_state, A, W, b):
    B, N, L, D = prop_state.shape
    H = prop_state.reshape(B, N, L * D)
    bias = b.reshape(1, D)

    out = pl.pallas_call(
        functools.partial(_body, B=B, N=N, L=L, D=D),
        in_specs=[
            pl.BlockSpec(memory_space=pltpu.MemorySpace.HBM),
            pl.BlockSpec(memory_space=pltpu.MemorySpace.HBM),
            pl.BlockSpec(memory_space=pltpu.MemorySpace.VMEM),
            pl.BlockSpec(memory_space=pltpu.MemorySpace.VMEM),
        ],
        out_specs=pl.BlockSpec(memory_space=pltpu.MemorySpace.HBM),
        out_shape=jax.ShapeDtypeStruct((B, N, L * D), jnp.float32),
        scratch_shapes=[
            pltpu.VMEM((NBUF, TM, N), jnp.float32),
            pltpu.VMEM((2, N, L * D), jnp.float32),
            pltpu.VMEM((2, N, L * D), jnp.float32),
            pltpu.VMEM((1, L * D), jnp.float32),
            pltpu.VMEM((2, TM, L * D), jnp.float32),
            pltpu.SemaphoreType.DMA((NBUF,)),
            pltpu.SemaphoreType.DMA((2,)),
            pltpu.SemaphoreType.DMA((2,)),
        ],
    )(A, H, W, bias)
    return out.reshape(B, N, L, D)


# speedup vs baseline: 1.0265x; 1.0265x over previous
import functools

import jax
import jax.numpy as jnp
from jax.experimental import pallas as pl
from jax.experimental.pallas import tpu as pltpu

TM = 1024


def _gcn_body(a_ref, h_ref, w_ref, bias_ref, o_ref, *, d):
    a = a_ref[0]
    ah = jnp.dot(a, h_ref[pl.program_id(0)], preferred_element_type=jnp.float32)
    ah2 = ah.reshape(-1, d)
    out = jax.lax.dot_general(
        ah2, w_ref[...], (((1,), (1,)), ((), ())),
        preferred_element_type=jnp.float32)
    out = jnp.maximum(out + bias_ref[...], 0.0)
    o_ref[0] = out.reshape(a.shape[0], -1)


def kernel(prop_state, A, W, b):
    B, N, L, D = prop_state.shape
    H = prop_state.reshape(B, N, L * D)
    bias = b.reshape(1, D)

    grid = (B, N // TM)
    out = pl.pallas_call(
        functools.partial(_gcn_body, d=D),
        grid=grid,
        in_specs=[
            pl.BlockSpec((1, TM, N), lambda bi, i: (bi, i, 0)),
            pl.BlockSpec((4, N, L * D), lambda bi, i: (0, 0, 0)),
            pl.BlockSpec((D, D), lambda bi, i: (0, 0)),
            pl.BlockSpec((1, D), lambda bi, i: (0, 0)),
        ],
        out_specs=pl.BlockSpec((1, TM, L * D), lambda bi, i: (bi, i, 0)),
        out_shape=jax.ShapeDtypeStruct((B, N, L * D), jnp.float32),
        compiler_params=pltpu.CompilerParams(
            dimension_semantics=("parallel", "parallel")),
    )(A, H, W, bias)
    return out.reshape(B, N, L, D)


# speedup vs baseline: 1.6118x; 1.5703x over previous
import functools

import jax
import jax.numpy as jnp
from jax.experimental import pallas as pl
from jax.experimental.pallas import tpu as pltpu

TM = 1024


def _gcn_body(a_ref, h_ref, w_ref, b_ref, o_ref, hw_ref, *, d, l):
    @pl.when(pl.program_id(1) == 0)
    def _():
        for ll in range(l):
            hw_ref[:, ll * d:(ll + 1) * d] = jax.lax.dot_general(
                h_ref[0, :, ll, :], w_ref[...],
                (((1,), (1,)), ((), ())),
                preferred_element_type=jnp.float32)

    out = jnp.dot(a_ref[0], hw_ref[...], preferred_element_type=jnp.float32)
    for ll in range(l):
        o_ref[0, :, ll, :] = jnp.maximum(
            out[:, ll * d:(ll + 1) * d] + b_ref[...], 0.0)


def kernel(prop_state, A, W, b):
    B, N, L, D = prop_state.shape
    bias = b.reshape(1, D)

    grid = (B, N // TM)
    return pl.pallas_call(
        functools.partial(_gcn_body, d=D, l=L),
        grid=grid,
        in_specs=[
            pl.BlockSpec((1, TM, N), lambda bi, i: (bi, i, 0)),
            pl.BlockSpec((1, N, L, D), lambda bi, i: (bi, 0, 0, 0)),
            pl.BlockSpec((D, D), lambda bi, i: (0, 0)),
            pl.BlockSpec((1, D), lambda bi, i: (0, 0)),
        ],
        out_specs=pl.BlockSpec((1, TM, L, D), lambda bi, i: (bi, i, 0, 0)),
        out_shape=jax.ShapeDtypeStruct((B, N, L, D), jnp.float32),
        scratch_shapes=[pltpu.VMEM((N, L * D), jnp.float32)],
        compiler_params=pltpu.CompilerParams(
            dimension_semantics=("arbitrary", "arbitrary")),
    )(A, prop_state, W, bias)
